# Initial kernel scaffold; baseline (speedup 1.0000x reference)
import jax
import jax.numpy as jnp
from jax.experimental import pallas as pl


def kernel(h, edge_index, WQ_w, WQ_b, WK_w, WK_b, WV_w, WV_b):
    raise NotImplementedError("write your pallas kernel here")


# speedup vs baseline: 10.3329x; 10.3329x over previous
import dataclasses
import functools

import jax
import jax.numpy as jnp
from jax import lax
from jax.experimental import pallas as pl
from jax.experimental.pallas import tpu as pltpu
from jax.experimental.pallas import tpu_sc as plsc

N_NODES = 10000
N_EDGES = 160000
HIDDEN = 256
HEADS = 8
DH = 32
HALF = 128
NC = 2
NS = 16
LANES = 16
CHUNK = 40
EDGES_PER_TILE = N_EDGES // NS
NCHUNK = EDGES_PER_TILE // CHUNK
NP = 10240
ROWS_PER_TILE = NP // NS
DEN_ROWS = NP // LANES
DEN_ROWS_PER_TILE = DEN_ROWS // NS
UNP_PARTS = 8
UNP_ROWS = ROWS_PER_TILE // UNP_PARTS
INV_SCALE = 1.0 / (DH ** 0.5)

_ROWB = 2000
_ROWB2 = 2048


def _qkv_body(h_ref, wq_ref, bq_ref, wk_ref, bk_ref, wv_ref, bv_ref,
              qt_ref, kt_ref, vt_ref):
    hb = h_ref[...]
    for w_ref, b_ref, o_ref in ((wq_ref, bq_ref, qt_ref),
                                (wk_ref, bk_ref, kt_ref),
                                (wv_ref, bv_ref, vt_ref)):
        for half in range(2):
            w = w_ref[half * HALF:(half + 1) * HALF, :]
            r = lax.dot_general(hb, w, (((1,), (1,)), ((), ())),
                                preferred_element_type=jnp.float32)
            o_ref[half] = r + b_ref[half]


def _qkv(h, WQ_w, bq2, WK_w, bk2, WV_w, bv2):
    tab = jax.ShapeDtypeStruct((NC, N_NODES, HALF), jnp.float32)
    wspec = pl.BlockSpec((HIDDEN, HIDDEN), lambda i: (0, 0))
    bspec = pl.BlockSpec((NC, HALF), lambda i: (0, 0))
    ospec = pl.BlockSpec((NC, _ROWB, HALF), lambda i: (0, i, 0))
    return pl.pallas_call(
        _qkv_body,
        grid=(N_NODES // _ROWB,),
        in_specs=[pl.BlockSpec((_ROWB, HIDDEN), lambda i: (i, 0)),
                  wspec, bspec, wspec, bspec, wspec, bspec],
        out_specs=[ospec, ospec, ospec],
        out_shape=[tab, tab, tab],
    )(h, WQ_w, bq2, WK_w, bk2, WV_w, bv2)


def _edge_body(qt_hbm, kt_hbm, vt_hbm, dst_hbm, src_hbm, zero_hbm,
               num_hbm, denf_hbm,
               dst_v0, dst_v1, src_v0, src_v1, hi_v0, hi_v1,
               qd_v0, qd_v1, ks_v0, ks_v1, vs_v0, vs_v1,
               row_v, den_row_v, unp_v, acc_num, acc_den, sem0, sem1):
    c = lax.axis_index("c")
    s = lax.axis_index("s")
    rowbase = s * ROWS_PER_TILE
    denbase = s * DEN_ROWS_PER_TILE
    lane = lax.iota(jnp.int32, LANES)
    rot8 = (lane + 8) & 15
    lane7 = lane & 7
    lanehi = jnp.right_shift(lane, 3)

    pltpu.sync_copy(zero_hbm.at[pl.ds(rowbase, ROWS_PER_TILE)],
                    acc_num.at[pl.ds(rowbase, ROWS_PER_TILE)])
    pltpu.sync_copy(zero_hbm.at[pl.ds(denbase, DEN_ROWS_PER_TILE)],
                    acc_den.at[pl.ds(denbase, DEN_ROWS_PER_TILE)])
    plsc.subcore_barrier()

    def load_idx(g, dst_v, src_v):
        base = s * EDGES_PER_TILE + g * CHUNK
        pltpu.sync_copy(dst_hbm.at[pl.ds(base, CHUNK)], dst_v)
        pltpu.sync_copy(src_hbm.at[pl.ds(base, CHUNK)], src_v)

    def gathers(dst_v, src_v, qd_v, ks_v, vs_v, sem):
        return (pltpu.make_async_copy(qt_hbm.at[c].at[dst_v], qd_v, sem),
                pltpu.make_async_copy(kt_hbm.at[c].at[src_v], ks_v, sem),
                pltpu.make_async_copy(vt_hbm.at[c].at[src_v], vs_v, sem))

    def start_gathers(*a):
        for cp_ in gathers(*a):
            cp_.start()

    def wait_gathers(*a):
        for cp_ in gathers(*a):
            cp_.wait()

    def do_chunk(dst_v, hi_v, qd_v, ks_v, vs_v):
        @pl.loop(0, CHUNK, step=LANES)
        def _shift(i):
            hi_v[pl.ds(i, LANES)] = jnp.right_shift(dst_v[pl.ds(i, LANES)], 4)

        @pl.loop(0, CHUNK)
        def _edge(e):
            den = None
            for hh in range(4):
                p0 = qd_v[e, pl.ds(hh * 32, LANES)] * ks_v[e, pl.ds(hh * 32, LANES)]
                p1 = (qd_v[e, pl.ds(hh * 32 + LANES, LANES)]
                      * ks_v[e, pl.ds(hh * 32 + LANES, LANES)])
                score = jnp.sum(p0 + p1) * INV_SCALE
                eb = jnp.exp(jnp.broadcast_to(score, (LANES,)))
                row_v[e, pl.ds(hh * 32, LANES)] = eb * vs_v[e, pl.ds(hh * 32, LANES)]
                row_v[e, pl.ds(hh * 32 + LANES, LANES)] = (
                    eb * vs_v[e, pl.ds(hh * 32 + LANES, LANES)])
                masked = jnp.where(lane == hh, eb, 0.0)
                den = masked if den is None else den + masked
            dvec = plsc.load_gather(dst_v, [jnp.broadcast_to(e, (LANES,))])
            bvec = dvec & 15
            gathered = den.at[lane7].get(mode="promise_in_bounds")
            for js in range(8):
                den_row_v[e, pl.ds(js * LANES, LANES)] = jnp.where(
                    (lanehi + 2 * js) == bvec, gathered, 0.0)

        pltpu.sync_copy(row_v, acc_num.at[dst_v], add=True)
        pltpu.sync_copy(den_row_v, acc_den.at[hi_v], add=True)

    bufs0 = (dst_v0, src_v0, qd_v0, ks_v0, vs_v0, sem0)
    bufs1 = (dst_v1, src_v1, qd_v1, ks_v1, vs_v1, sem1)
    load_idx(0, dst_v0, src_v0)
    start_gathers(*bufs0)

    @pl.loop(0, NCHUNK, step=2)
    def _chunk(g):
        load_idx(g + 1, dst_v1, src_v1)
        start_gathers(*bufs1)
        wait_gathers(*bufs0)
        do_chunk(dst_v0, hi_v0, qd_v0, ks_v0, vs_v0)

        @pl.when(g + 2 < NCHUNK)
        def _prefetch():
            load_idx(g + 2, dst_v0, src_v0)
            start_gathers(*bufs0)
        wait_gathers(*bufs1)
        do_chunk(dst_v1, hi_v1, qd_v1, ks_v1, vs_v1)

    plsc.subcore_barrier()
    pltpu.sync_copy(acc_num.at[pl.ds(rowbase, ROWS_PER_TILE)],
                    num_hbm.at[c].at[pl.ds(rowbase, ROWS_PER_TILE)])

    pltpu.sync_copy(acc_den.at[pl.ds(denbase, DEN_ROWS_PER_TILE)], qd_v0)

    @pl.loop(0, UNP_PARTS)
    def _part(p):
        @pl.loop(0, DEN_ROWS_PER_TILE // UNP_PARTS)
        def _unpack(rr):
            r = p * (DEN_ROWS_PER_TILE // UNP_PARTS) + rr
            for jv in range(8):
                srcv = qd_v0[r, pl.ds(jv * LANES, LANES)]
                rot = srcv.at[rot8].get(mode="promise_in_bounds")
                unp_v[rr * LANES + 2 * jv, pl.ds(0, LANES)] = jnp.where(
                    lane < 8, srcv, 0.0)
                unp_v[rr * LANES + 2 * jv + 1, pl.ds(0, LANES)] = jnp.where(
                    lane < 8, rot, 0.0)
        pltpu.sync_copy(unp_v,
                        denf_hbm.at[c].at[pl.ds(rowbase + p * UNP_ROWS, UNP_ROWS)])


def _edges(qt, kt, vt, dst, src, zeros):
    mesh = plsc.VectorSubcoreMesh(core_axis_name="c", subcore_axis_name="s")
    cp = pltpu.CompilerParams()
    if "needs_layout_passes" in pltpu.CompilerParams.__dataclass_fields__:
        cp = dataclasses.replace(cp, needs_layout_passes=False)
    if "use_tc_tiling_on_sc" in pltpu.CompilerParams.__dataclass_fields__:
        cp = dataclasses.replace(cp, use_tc_tiling_on_sc=False)
    fn = functools.partial(
        pl.kernel,
        mesh=mesh,
        compiler_params=cp,
        out_type=[jax.ShapeDtypeStruct((NC, NP, HALF), jnp.float32),
                  jax.ShapeDtypeStruct((NC, NP, LANES), jnp.float32)],
        scratch_types=[
            pltpu.VMEM((CHUNK,), jnp.int32),
            pltpu.VMEM((CHUNK,), jnp.int32),
            pltpu.VMEM((CHUNK,), jnp.int32),
            pltpu.VMEM((CHUNK,), jnp.int32),
            pltpu.VMEM((CHUNK,), jnp.int32),
            pltpu.VMEM((CHUNK,), jnp.int32),
            pltpu.VMEM((CHUNK, HALF), jnp.float32),
            pltpu.VMEM((CHUNK, HALF), jnp.float32),
            pltpu.VMEM((CHUNK, HALF), jnp.float32),
            pltpu.VMEM((CHUNK, HALF), jnp.float32),
            pltpu.VMEM((CHUNK, HALF), jnp.float32),
            pltpu.VMEM((CHUNK, HALF), jnp.float32),
            pltpu.VMEM((CHUNK, HALF), jnp.float32),
            pltpu.VMEM((CHUNK, HALF), jnp.float32),
            pltpu.VMEM((UNP_ROWS, LANES), jnp.float32),
            pltpu.VMEM_SHARED((NP, HALF), jnp.float32),
            pltpu.VMEM_SHARED((DEN_ROWS, HALF), jnp.float32),
            pltpu.SemaphoreType.DMA,
            pltpu.SemaphoreType.DMA,
        ],
    )(_edge_body)
    return fn(qt, kt, vt, dst, src, zeros)


def _norm_body(num_ref, den_ref, out_ref):
    for c in range(NC):
        for hh in range(4):
            numer = num_ref[c, :, hh * 32:(hh + 1) * 32]
            den = den_ref[c, :, hh:hh + 1]
            out_ref[:, (c * 4 + hh) * 32:(c * 4 + hh + 1) * 32] = (
                numer / jnp.maximum(den, 1e-16))


def _norm(num, denf):
    return pl.pallas_call(
        _norm_body,
        grid=(NP // _ROWB2,),
        in_specs=[pl.BlockSpec((NC, _ROWB2, HALF), lambda i: (0, i, 0)),
                  pl.BlockSpec((NC, _ROWB2, LANES), lambda i: (0, i, 0))],
        out_specs=pl.BlockSpec((_ROWB2, HIDDEN), lambda i: (i, 0)),
        out_shape=jax.ShapeDtypeStruct((NP, HIDDEN), jnp.float32),
    )(num, denf)


def kernel(h, edge_index, WQ_w, WQ_b, WK_w, WK_b, WV_w, WV_b):
    src = edge_index[0].astype(jnp.int32)
    dst = edge_index[1].astype(jnp.int32)
    bq2 = WQ_b.reshape(NC, HALF)
    bk2 = WK_b.reshape(NC, HALF)
    bv2 = WV_b.reshape(NC, HALF)
    qt, kt, vt = _qkv(h, WQ_w, bq2, WK_w, bk2, WV_w, bv2)
    zeros = jnp.zeros((NP, HALF), jnp.float32)
    num, denf = _edges(qt, kt, vt, dst, src, zeros)
    out = _norm(num, denf)
    return out[:N_NODES].reshape(N_NODES, HEADS, DH)


# speedup vs baseline: 36.4056x; 3.5233x over previous
import dataclasses
import functools

import jax
import jax.numpy as jnp
from jax import lax
from jax.experimental import pallas as pl
from jax.experimental.pallas import tpu as pltpu
from jax.experimental.pallas import tpu_sc as plsc

N_NODES = 10000
N_EDGES = 160000
HIDDEN = 256
HEADS = 8
DH = 32
HALF = 128
NC = 2
NS = 16
LANES = 16
CHUNK = 40
EDGES_PER_TILE = N_EDGES // NS
NCHUNK = EDGES_PER_TILE // CHUNK
NP = 10240
ROWS_PER_TILE = NP // NS
DEN_ROWS = NP // LANES
DEN_ROWS_PER_TILE = DEN_ROWS // NS
UNP_PARTS = 8
UNP_ROWS = ROWS_PER_TILE // UNP_PARTS
INV_SCALE = 1.0 / (DH ** 0.5)

_ROWB = 2000
_ROWB2 = 2048


def _qkv_body(h_ref, wq_ref, bq_ref, wk_ref, bk_ref, wv_ref, bv_ref,
              qt_ref, kt_ref, vt_ref):
    hb = h_ref[...]
    for w_ref, b_ref, o_ref in ((wq_ref, bq_ref, qt_ref),
                                (wk_ref, bk_ref, kt_ref),
                                (wv_ref, bv_ref, vt_ref)):
        for half in range(2):
            w = w_ref[half * HALF:(half + 1) * HALF, :]
            r = lax.dot_general(hb, w, (((1,), (1,)), ((), ())),
                                preferred_element_type=jnp.float32)
            o_ref[half] = r + b_ref[half]


def _qkv(h, WQ_w, bq2, WK_w, bk2, WV_w, bv2):
    tab = jax.ShapeDtypeStruct((NC, N_NODES, HALF), jnp.float32)
    wspec = pl.BlockSpec((HIDDEN, HIDDEN), lambda i: (0, 0))
    bspec = pl.BlockSpec((NC, HALF), lambda i: (0, 0))
    ospec = pl.BlockSpec((NC, _ROWB, HALF), lambda i: (0, i, 0))
    return pl.pallas_call(
        _qkv_body,
        grid=(N_NODES // _ROWB,),
        in_specs=[pl.BlockSpec((_ROWB, HIDDEN), lambda i: (i, 0)),
                  wspec, bspec, wspec, bspec, wspec, bspec],
        out_specs=[ospec, ospec, ospec],
        out_shape=[tab, tab, tab],
    )(h, WQ_w, bq2, WK_w, bk2, WV_w, bv2)


def _edge_body(qt_hbm, kt_hbm, vt_hbm, dst_hbm, src_hbm, zero_hbm, zden_hbm,
               num_hbm, denf_hbm,
               dst_v0, dst_v1, src_v0, src_v1,
               qd_v0, qd_v1, ks_v0, ks_v1, vs_v0, vs_v1,
               row_v, den_row_v, acc_num, acc_den, sem0, sem1):
    c = lax.axis_index("c")
    s = lax.axis_index("s")
    rowbase = s * ROWS_PER_TILE
    lane = lax.iota(jnp.int32, LANES)

    pltpu.sync_copy(zero_hbm.at[pl.ds(rowbase, ROWS_PER_TILE)],
                    acc_num.at[pl.ds(rowbase, ROWS_PER_TILE)])
    pltpu.sync_copy(zden_hbm.at[pl.ds(rowbase, ROWS_PER_TILE)],
                    acc_den.at[pl.ds(rowbase, ROWS_PER_TILE)])
    plsc.subcore_barrier()

    def load_idx(g, dst_v, src_v):
        base = s * EDGES_PER_TILE + g * CHUNK
        pltpu.sync_copy(dst_hbm.at[pl.ds(base, CHUNK)], dst_v)
        pltpu.sync_copy(src_hbm.at[pl.ds(base, CHUNK)], src_v)

    def gathers(dst_v, src_v, qd_v, ks_v, vs_v, sem):
        return (pltpu.make_async_copy(qt_hbm.at[c].at[dst_v], qd_v, sem),
                pltpu.make_async_copy(kt_hbm.at[c].at[src_v], ks_v, sem),
                pltpu.make_async_copy(vt_hbm.at[c].at[src_v], vs_v, sem))

    def start_gathers(*a):
        for cp_ in gathers(*a):
            cp_.start()

    def wait_gathers(*a):
        for cp_ in gathers(*a):
            cp_.wait()

    def do_chunk(dst_v, qd_v, ks_v, vs_v):
        @plsc.parallel_loop(0, CHUNK, unroll=2)
        def _edge(e):
            den = None
            for hh in range(4):
                p0 = qd_v[e, pl.ds(hh * 32, LANES)] * ks_v[e, pl.ds(hh * 32, LANES)]
                p1 = (qd_v[e, pl.ds(hh * 32 + LANES, LANES)]
                      * ks_v[e, pl.ds(hh * 32 + LANES, LANES)])
                score = jnp.sum(p0 + p1) * INV_SCALE
                eb = jnp.exp(jnp.broadcast_to(score, (LANES,)))
                row_v[e, pl.ds(hh * 32, LANES)] = eb * vs_v[e, pl.ds(hh * 32, LANES)]
                row_v[e, pl.ds(hh * 32 + LANES, LANES)] = (
                    eb * vs_v[e, pl.ds(hh * 32 + LANES, LANES)])
                masked = jnp.where(lane == hh, eb, 0.0)
                den = masked if den is None else den + masked
            den_row_v[e, pl.ds(0, LANES)] = den

        pltpu.sync_copy(row_v, acc_num.at[dst_v], add=True)
        pltpu.sync_copy(den_row_v, acc_den.at[dst_v], add=True)

    bufs0 = (dst_v0, src_v0, qd_v0, ks_v0, vs_v0, sem0)
    bufs1 = (dst_v1, src_v1, qd_v1, ks_v1, vs_v1, sem1)
    load_idx(0, dst_v0, src_v0)
    start_gathers(*bufs0)

    @pl.loop(0, NCHUNK, step=2)
    def _chunk(g):
        load_idx(g + 1, dst_v1, src_v1)
        start_gathers(*bufs1)
        wait_gathers(*bufs0)
        do_chunk(dst_v0, qd_v0, ks_v0, vs_v0)

        @pl.when(g + 2 < NCHUNK)
        def _prefetch():
            load_idx(g + 2, dst_v0, src_v0)
            start_gathers(*bufs0)
        wait_gathers(*bufs1)
        do_chunk(dst_v1, qd_v1, ks_v1, vs_v1)

    plsc.subcore_barrier()
    pltpu.sync_copy(acc_num.at[pl.ds(rowbase, ROWS_PER_TILE)],
                    num_hbm.at[c].at[pl.ds(rowbase, ROWS_PER_TILE)])
    pltpu.sync_copy(acc_den.at[pl.ds(rowbase, ROWS_PER_TILE)],
                    denf_hbm.at[c].at[pl.ds(rowbase, ROWS_PER_TILE)])


def _edges(qt, kt, vt, dst, src, zeros):
    mesh = plsc.VectorSubcoreMesh(core_axis_name="c", subcore_axis_name="s")
    cp = pltpu.CompilerParams()
    if "needs_layout_passes" in pltpu.CompilerParams.__dataclass_fields__:
        cp = dataclasses.replace(cp, needs_layout_passes=False)
    if "use_tc_tiling_on_sc" in pltpu.CompilerParams.__dataclass_fields__:
        cp = dataclasses.replace(cp, use_tc_tiling_on_sc=False)
    fn = functools.partial(
        pl.kernel,
        mesh=mesh,
        compiler_params=cp,
        out_type=[jax.ShapeDtypeStruct((NC, NP, HALF), jnp.float32),
                  jax.ShapeDtypeStruct((NC, NP, LANES), jnp.float32)],
        scratch_types=[
            pltpu.VMEM((CHUNK,), jnp.int32),
            pltpu.VMEM((CHUNK,), jnp.int32),
            pltpu.VMEM((CHUNK,), jnp.int32),
            pltpu.VMEM((CHUNK,), jnp.int32),
            pltpu.VMEM((CHUNK, HALF), jnp.float32),
            pltpu.VMEM((CHUNK, HALF), jnp.float32),
            pltpu.VMEM((CHUNK, HALF), jnp.float32),
            pltpu.VMEM((CHUNK, HALF), jnp.float32),
            pltpu.VMEM((CHUNK, HALF), jnp.float32),
            pltpu.VMEM((CHUNK, HALF), jnp.float32),
            pltpu.VMEM((CHUNK, HALF), jnp.float32),
            pltpu.VMEM((CHUNK, LANES), jnp.float32),
            pltpu.VMEM_SHARED((NP, HALF), jnp.float32),
            pltpu.VMEM_SHARED((NP, LANES), jnp.float32),
            pltpu.SemaphoreType.DMA,
            pltpu.SemaphoreType.DMA,
        ],
    )(_edge_body)
    return fn(qt, kt, vt, dst, src, zeros, jnp.zeros((NP, LANES), jnp.float32))


def _norm_body(num_ref, den_ref, out_ref):
    for c in range(NC):
        for hh in range(4):
            numer = num_ref[c, :, hh * 32:(hh + 1) * 32]
            den = den_ref[c, :, hh:hh + 1]
            out_ref[:, (c * 4 + hh) * 32:(c * 4 + hh + 1) * 32] = (
                numer / jnp.maximum(den, 1e-16))


def _norm(num, denf):
    return pl.pallas_call(
        _norm_body,
        grid=(NP // _ROWB2,),
        in_specs=[pl.BlockSpec((NC, _ROWB2, HALF), lambda i: (0, i, 0)),
                  pl.BlockSpec((NC, _ROWB2, LANES), lambda i: (0, i, 0))],
        out_specs=pl.BlockSpec((_ROWB2, HIDDEN), lambda i: (i, 0)),
        out_shape=jax.ShapeDtypeStruct((NP, HIDDEN), jnp.float32),
    )(num, denf)


def kernel(h, edge_index, WQ_w, WQ_b, WK_w, WK_b, WV_w, WV_b):
    src = edge_index[0].astype(jnp.int32)
    dst = edge_index[1].astype(jnp.int32)
    bq2 = WQ_b.reshape(NC, HALF)
    bk2 = WK_b.reshape(NC, HALF)
    bv2 = WV_b.reshape(NC, HALF)
    qt, kt, vt = _qkv(h, WQ_w, bq2, WK_w, bk2, WV_w, bv2)
    zeros = jnp.zeros((NP, HALF), jnp.float32)
    num, denf = _edges(qt, kt, vt, dst, src, zeros)
    out = _norm(num, denf)
    return out[:N_NODES].reshape(N_NODES, HEADS, DH)


# speedup vs baseline: 36.4632x; 1.0016x over previous
import dataclasses
import functools

import jax
import jax.numpy as jnp
from jax import lax
from jax.experimental import pallas as pl
from jax.experimental.pallas import tpu as pltpu
from jax.experimental.pallas import tpu_sc as plsc

N_NODES = 10000
N_EDGES = 160000
HIDDEN = 256
HEADS = 8
DH = 32
HALF = 128
NC = 2
NS = 16
LANES = 16
CHUNK = 40
EDGES_PER_TILE = N_EDGES // NS
NCHUNK = EDGES_PER_TILE // CHUNK
NP = 10240
ROWS_PER_TILE = NP // NS
DEN_ROWS = NP // LANES
DEN_ROWS_PER_TILE = DEN_ROWS // NS
UNP_PARTS = 8
UNP_ROWS = ROWS_PER_TILE // UNP_PARTS
INV_SCALE = 1.0 / (DH ** 0.5)

_ROWB = 2000
_ROWB2 = 2048


def _qkv_body(h_ref, wq_ref, bq_ref, wk_ref, bk_ref, wv_ref, bv_ref,
              qt_ref, kvt_ref):
    hb = h_ref[...]

    def proj(w_ref, b_ref, half):
        w = w_ref[half * HALF:(half + 1) * HALF, :]
        r = lax.dot_general(hb, w, (((1,), (1,)), ((), ())),
                            preferred_element_type=jnp.float32)
        return r + b_ref[half]

    for half in range(2):
        qt_ref[half] = proj(wq_ref, bq_ref, half)
        kvt_ref[half, :, 0:HALF] = proj(wk_ref, bk_ref, half)
        kvt_ref[half, :, HALF:2 * HALF] = proj(wv_ref, bv_ref, half)


def _qkv(h, WQ_w, bq2, WK_w, bk2, WV_w, bv2):
    wspec = pl.BlockSpec((HIDDEN, HIDDEN), lambda i: (0, 0))
    bspec = pl.BlockSpec((NC, HALF), lambda i: (0, 0))
    return pl.pallas_call(
        _qkv_body,
        grid=(N_NODES // _ROWB,),
        in_specs=[pl.BlockSpec((_ROWB, HIDDEN), lambda i: (i, 0)),
                  wspec, bspec, wspec, bspec, wspec, bspec],
        out_specs=[pl.BlockSpec((NC, _ROWB, HALF), lambda i: (0, i, 0)),
                   pl.BlockSpec((NC, _ROWB, 2 * HALF), lambda i: (0, i, 0))],
        out_shape=[jax.ShapeDtypeStruct((NC, N_NODES, HALF), jnp.float32),
                   jax.ShapeDtypeStruct((NC, N_NODES, 2 * HALF), jnp.float32)],
    )(h, WQ_w, bq2, WK_w, bk2, WV_w, bv2)


def _edge_body(qt_hbm, kvt_hbm, dst_hbm, src_hbm, zero_hbm, zden_hbm,
               num_hbm, denf_hbm,
               dst_v0, dst_v1, src_v0, src_v1,
               qd_v0, qd_v1, kv_v0, kv_v1,
               row_v, den_row_v, acc_num, acc_den, sem0, sem1):
    c = lax.axis_index("c")
    s = lax.axis_index("s")
    rowbase = s * ROWS_PER_TILE
    lane = lax.iota(jnp.int32, LANES)

    pltpu.sync_copy(zero_hbm.at[pl.ds(rowbase, ROWS_PER_TILE)],
                    acc_num.at[pl.ds(rowbase, ROWS_PER_TILE)])
    pltpu.sync_copy(zden_hbm.at[pl.ds(rowbase, ROWS_PER_TILE)],
                    acc_den.at[pl.ds(rowbase, ROWS_PER_TILE)])
    plsc.subcore_barrier()

    def load_idx(g, dst_v, src_v):
        base = s * EDGES_PER_TILE + g * CHUNK
        pltpu.sync_copy(dst_hbm.at[pl.ds(base, CHUNK)], dst_v)
        pltpu.sync_copy(src_hbm.at[pl.ds(base, CHUNK)], src_v)

    def gathers(dst_v, src_v, qd_v, kv_v, sem):
        return (pltpu.make_async_copy(qt_hbm.at[c].at[dst_v], qd_v, sem),
                pltpu.make_async_copy(kvt_hbm.at[c].at[src_v], kv_v, sem))

    def start_gathers(*a):
        for cp_ in gathers(*a):
            cp_.start()

    def wait_gathers(*a):
        for cp_ in gathers(*a):
            cp_.wait()

    def do_chunk(dst_v, qd_v, kv_v):
        @plsc.parallel_loop(0, CHUNK, unroll=4)
        def _edge(e):
            den = None
            for hh in range(4):
                p0 = qd_v[e, pl.ds(hh * 32, LANES)] * kv_v[e, pl.ds(hh * 32, LANES)]
                p1 = (qd_v[e, pl.ds(hh * 32 + LANES, LANES)]
                      * kv_v[e, pl.ds(hh * 32 + LANES, LANES)])
                score = jnp.sum(p0 + p1) * INV_SCALE
                eb = jnp.exp(jnp.broadcast_to(score, (LANES,)))
                row_v[e, pl.ds(hh * 32, LANES)] = (
                    eb * kv_v[e, pl.ds(HALF + hh * 32, LANES)])
                row_v[e, pl.ds(hh * 32 + LANES, LANES)] = (
                    eb * kv_v[e, pl.ds(HALF + hh * 32 + LANES, LANES)])
                masked = jnp.where(lane == hh, eb, 0.0)
                den = masked if den is None else den + masked
            den_row_v[e, pl.ds(0, LANES)] = den

        pltpu.sync_copy(row_v, acc_num.at[dst_v], add=True)
        pltpu.sync_copy(den_row_v, acc_den.at[dst_v], add=True)

    bufs0 = (dst_v0, src_v0, qd_v0, kv_v0, sem0)
    bufs1 = (dst_v1, src_v1, qd_v1, kv_v1, sem1)
    load_idx(0, dst_v0, src_v0)
    start_gathers(*bufs0)

    @pl.loop(0, NCHUNK, step=2)
    def _chunk(g):
        load_idx(g + 1, dst_v1, src_v1)
        start_gathers(*bufs1)
        wait_gathers(*bufs0)
        do_chunk(dst_v0, qd_v0, kv_v0)

        @pl.when(g + 2 < NCHUNK)
        def _prefetch():
            load_idx(g + 2, dst_v0, src_v0)
            start_gathers(*bufs0)
        wait_gathers(*bufs1)
        do_chunk(dst_v1, qd_v1, kv_v1)

    plsc.subcore_barrier()
    pltpu.sync_copy(acc_num.at[pl.ds(rowbase, ROWS_PER_TILE)],
                    num_hbm.at[c].at[pl.ds(rowbase, ROWS_PER_TILE)])
    pltpu.sync_copy(acc_den.at[pl.ds(rowbase, ROWS_PER_TILE)],
                    denf_hbm.at[c].at[pl.ds(rowbase, ROWS_PER_TILE)])


def _edges(qt, kvt, dst, src, zeros):
    mesh = plsc.VectorSubcoreMesh(core_axis_name="c", subcore_axis_name="s")
    cp = pltpu.CompilerParams()
    if "needs_layout_passes" in pltpu.CompilerParams.__dataclass_fields__:
        cp = dataclasses.replace(cp, needs_layout_passes=False)
    if "use_tc_tiling_on_sc" in pltpu.CompilerParams.__dataclass_fields__:
        cp = dataclasses.replace(cp, use_tc_tiling_on_sc=False)
    fn = functools.partial(
        pl.kernel,
        mesh=mesh,
        compiler_params=cp,
        out_type=[jax.ShapeDtypeStruct((NC, NP, HALF), jnp.float32),
                  jax.ShapeDtypeStruct((NC, NP, LANES), jnp.float32)],
        scratch_types=[
            pltpu.VMEM((CHUNK,), jnp.int32),
            pltpu.VMEM((CHUNK,), jnp.int32),
            pltpu.VMEM((CHUNK,), jnp.int32),
            pltpu.VMEM((CHUNK,), jnp.int32),
            pltpu.VMEM((CHUNK, HALF), jnp.float32),
            pltpu.VMEM((CHUNK, HALF), jnp.float32),
            pltpu.VMEM((CHUNK, 2 * HALF), jnp.float32),
            pltpu.VMEM((CHUNK, 2 * HALF), jnp.float32),
            pltpu.VMEM((CHUNK, HALF), jnp.float32),
            pltpu.VMEM((CHUNK, LANES), jnp.float32),
            pltpu.VMEM_SHARED((NP, HALF), jnp.float32),
            pltpu.VMEM_SHARED((NP, LANES), jnp.float32),
            pltpu.SemaphoreType.DMA,
            pltpu.SemaphoreType.DMA,
        ],
    )(_edge_body)
    return fn(qt, kvt, dst, src, zeros, jnp.zeros((NP, LANES), jnp.float32))


def _norm_body(num_ref, den_ref, out_ref):
    for c in range(NC):
        for hh in range(4):
            numer = num_ref[c, :, hh * 32:(hh + 1) * 32]
            den = den_ref[c, :, hh:hh + 1]
            out_ref[:, (c * 4 + hh) * 32:(c * 4 + hh + 1) * 32] = (
                numer / jnp.maximum(den, 1e-16))


def _norm(num, denf):
    return pl.pallas_call(
        _norm_body,
        grid=(NP // _ROWB2,),
        in_specs=[pl.BlockSpec((NC, _ROWB2, HALF), lambda i: (0, i, 0)),
                  pl.BlockSpec((NC, _ROWB2, LANES), lambda i: (0, i, 0))],
        out_specs=pl.BlockSpec((_ROWB2, HIDDEN), lambda i: (i, 0)),
        out_shape=jax.ShapeDtypeStruct((NP, HIDDEN), jnp.float32),
    )(num, denf)


def kernel(h, edge_index, WQ_w, WQ_b, WK_w, WK_b, WV_w, WV_b):
    src = edge_index[0].astype(jnp.int32)
    dst = edge_index[1].astype(jnp.int32)
    bq2 = WQ_b.reshape(NC, HALF)
    bk2 = WK_b.reshape(NC, HALF)
    bv2 = WV_b.reshape(NC, HALF)
    qt, kvt = _qkv(h, WQ_w, bq2, WK_w, bk2, WV_w, bv2)
    zeros = jnp.zeros((NP, HALF), jnp.float32)
    num, denf = _edges(qt, kvt, dst, src, zeros)
    out = _norm(num, denf)
    return out[:N_NODES].reshape(N_NODES, HEADS, DH)


# speedup vs baseline: 40.9405x; 1.1228x over previous
import dataclasses
import functools

import jax
import jax.numpy as jnp
from jax import lax
from jax.experimental import pallas as pl
from jax.experimental.pallas import tpu as pltpu
from jax.experimental.pallas import tpu_sc as plsc

N_NODES = 10000
N_EDGES = 160000
HIDDEN = 256
HEADS = 8
DH = 32
HALF = 128
ACCW = 144
NC = 2
NS = 16
LANES = 16
CHUNK = 32
EP = 160256
EDGES_PER_TILE = EP // NS
NCHUNK = EDGES_PER_TILE // CHUNK
NP = 10240
ROWS_PER_TILE = NP // NS
INV_SCALE = 1.0 / (DH ** 0.5)

_ROWB = 2000
_ROWB2 = 2048


def _qkv_body(h_ref, wq_ref, bq_ref, wk_ref, bk_ref, wv_ref, bv_ref,
              qt_ref, kvt_ref):
    hb = h_ref[...]

    def proj(w_ref, b_ref, half):
        w = w_ref[half * HALF:(half + 1) * HALF, :]
        r = lax.dot_general(hb, w, (((1,), (1,)), ((), ())),
                            preferred_element_type=jnp.float32)
        return r + b_ref[half]

    for half in range(2):
        qt_ref[half] = proj(wq_ref, bq_ref, half)
        kvt_ref[half, :, 0:HALF] = proj(wk_ref, bk_ref, half)
        kvt_ref[half, :, HALF:2 * HALF] = proj(wv_ref, bv_ref, half)


def _qkv(h, WQ_w, bq2, WK_w, bk2, WV_w, bv2):
    wspec = pl.BlockSpec((HIDDEN, HIDDEN), lambda i: (0, 0))
    bspec = pl.BlockSpec((NC, HALF), lambda i: (0, 0))
    return pl.pallas_call(
        _qkv_body,
        grid=(N_NODES // _ROWB,),
        in_specs=[pl.BlockSpec((_ROWB, HIDDEN), lambda i: (i, 0)),
                  wspec, bspec, wspec, bspec, wspec, bspec],
        out_specs=[pl.BlockSpec((NC, _ROWB, HALF), lambda i: (0, i, 0)),
                   pl.BlockSpec((NC, _ROWB, 2 * HALF), lambda i: (0, i, 0))],
        out_shape=[jax.ShapeDtypeStruct((NC, N_NODES, HALF), jnp.float32),
                   jax.ShapeDtypeStruct((NC, N_NODES, 2 * HALF), jnp.float32)],
    )(h, WQ_w, bq2, WK_w, bk2, WV_w, bv2)


def _edge_body(qt_hbm, kvt_hbm, ei_hbm, zero_hbm, acc_hbm,
               ei_v0, ei_v1, dsc_v0, dsc_v1, gcl_v0, gcl_v1,
               qd_v0, qd_v1, kv_v0, kv_v1, row_v0, row_v1,
               acc_sh, semi0, semi1, semg0, semg1, sems0, sems1):
    c = lax.axis_index("c")
    s = lax.axis_index("s")
    rowbase = s * ROWS_PER_TILE
    lane = lax.iota(jnp.int32, LANES)

    pltpu.sync_copy(zero_hbm.at[pl.ds(rowbase, ROWS_PER_TILE)],
                    acc_sh.at[pl.ds(rowbase, ROWS_PER_TILE)])
    plsc.subcore_barrier()

    EI = (ei_v0, ei_v1)
    DSC = (dsc_v0, dsc_v1)
    GCL = (gcl_v0, gcl_v1)
    QD = (qd_v0, qd_v1)
    KV = (kv_v0, kv_v1)
    ROW = (row_v0, row_v1)
    SEMI = (semi0, semi1)
    SEMG = (semg0, semg1)
    SEMS = (sems0, sems1)

    def idx_copy(k, p):
        base = s * EDGES_PER_TILE + k * CHUNK
        return pltpu.make_async_copy(ei_hbm.at[:, pl.ds(base, CHUNK)],
                                     EI[p], SEMI[p])

    def gather_copies(p):
        return (pltpu.make_async_copy(qt_hbm.at[c].at[GCL[p]], QD[p], SEMG[p]),
                pltpu.make_async_copy(kvt_hbm.at[c].at[EI[p].at[0]], KV[p],
                                      SEMG[p]))

    def scatter_copy(p):
        return pltpu.make_async_copy(ROW[p], acc_sh.at[DSC[p]], SEMS[p])

    def build_gcl(p):
        for i in range(CHUNK // LANES):
            GCL[p][pl.ds(i * LANES, LANES)] = jnp.minimum(
                EI[p][1, pl.ds(i * LANES, LANES)], N_NODES - 1)

    def build_dsc(p):
        for i in range(CHUNK // LANES):
            DSC[p][pl.ds(i * LANES, LANES)] = EI[p][1, pl.ds(i * LANES, LANES)]

    def compute(p):
        qd_v, kv_v, row_v = QD[p], KV[p], ROW[p]

        @plsc.parallel_loop(0, CHUNK, unroll=4)
        def _edge(e):
            den = None
            for hh in range(4):
                p0 = qd_v[e, pl.ds(hh * 32, LANES)] * kv_v[e, pl.ds(hh * 32, LANES)]
                p1 = (qd_v[e, pl.ds(hh * 32 + LANES, LANES)]
                      * kv_v[e, pl.ds(hh * 32 + LANES, LANES)])
                score = jnp.sum(p0 + p1) * INV_SCALE
                eb = jnp.exp(jnp.broadcast_to(score, (LANES,)))
                row_v[e, pl.ds(hh * 32, LANES)] = (
                    eb * kv_v[e, pl.ds(HALF + hh * 32, LANES)])
                row_v[e, pl.ds(hh * 32 + LANES, LANES)] = (
                    eb * kv_v[e, pl.ds(HALF + hh * 32 + LANES, LANES)])
                masked = jnp.where(lane == hh, eb, 0.0)
                den = masked if den is None else den + masked
            row_v[e, pl.ds(HALF, LANES)] = den

    def process(k, p, q, first, last):
        gather_copies(p)[0].wait()
        gather_copies(p)[1].wait()
        build_dsc(p)
        @pl.when(k + 2 < NCHUNK)
        def _():
            idx_copy(k + 2, p).start()
        @pl.when(k + 1 < NCHUNK)
        def _():
            idx_copy(k + 1, q).wait()
            build_gcl(q)
            for cp_ in gather_copies(q):
                cp_.start()
        if not first:
            @pl.when(k >= 2)
            def _():
                scatter_copy(p).wait()
        compute(p)
        scatter_copy(p).start(add=True)

    idx_copy(0, 0).start()
    idx_copy(1, 1).start()
    idx_copy(0, 0).wait()
    build_gcl(0)
    for cp_ in gather_copies(0):
        cp_.start()

    process(0, 0, 1, True, False)
    process(1, 1, 0, True, False)

    @pl.loop(2, NCHUNK - 1, step=2)
    def _chunk(g):
        process(g, 0, 1, False, False)
        process(g + 1, 1, 0, False, False)

    process(NCHUNK - 1, 0, 1, False, True)

    scatter_copy(1).wait()
    scatter_copy(0).wait()
    plsc.subcore_barrier()
    pltpu.sync_copy(acc_sh.at[pl.ds(rowbase, ROWS_PER_TILE)],
                    acc_hbm.at[c].at[pl.ds(rowbase, ROWS_PER_TILE)])


def _edges(qt, kvt, ei, zeros):
    mesh = plsc.VectorSubcoreMesh(core_axis_name="c", subcore_axis_name="s")
    cp = pltpu.CompilerParams()
    if "needs_layout_passes" in pltpu.CompilerParams.__dataclass_fields__:
        cp = dataclasses.replace(cp, needs_layout_passes=False)
    if "use_tc_tiling_on_sc" in pltpu.CompilerParams.__dataclass_fields__:
        cp = dataclasses.replace(cp, use_tc_tiling_on_sc=False)
    fn = functools.partial(
        pl.kernel,
        mesh=mesh,
        compiler_params=cp,
        out_type=jax.ShapeDtypeStruct((NC, NP, ACCW), jnp.float32),
        scratch_types=[
            pltpu.VMEM((2, CHUNK), jnp.int32),
            pltpu.VMEM((2, CHUNK), jnp.int32),
            pltpu.VMEM((CHUNK,), jnp.int32),
            pltpu.VMEM((CHUNK,), jnp.int32),
            pltpu.VMEM((CHUNK,), jnp.int32),
            pltpu.VMEM((CHUNK,), jnp.int32),
            pltpu.VMEM((CHUNK, HALF), jnp.float32),
            pltpu.VMEM((CHUNK, HALF), jnp.float32),
            pltpu.VMEM((CHUNK, 2 * HALF), jnp.float32),
            pltpu.VMEM((CHUNK, 2 * HALF), jnp.float32),
            pltpu.VMEM((CHUNK, ACCW), jnp.float32),
            pltpu.VMEM((CHUNK, ACCW), jnp.float32),
            pltpu.VMEM_SHARED((NP, ACCW), jnp.float32),
            pltpu.SemaphoreType.DMA,
            pltpu.SemaphoreType.DMA,
            pltpu.SemaphoreType.DMA,
            pltpu.SemaphoreType.DMA,
            pltpu.SemaphoreType.DMA,
            pltpu.SemaphoreType.DMA,
        ],
    )(_edge_body)
    return fn(qt, kvt, ei, zeros)


def _norm_body(acc_ref, out_ref):
    for c in range(NC):
        for hh in range(4):
            numer = acc_ref[c, :, hh * 32:(hh + 1) * 32]
            den = acc_ref[c, :, HALF + hh:HALF + hh + 1]
            out_ref[:, (c * 4 + hh) * 32:(c * 4 + hh + 1) * 32] = (
                numer / jnp.maximum(den, 1e-16))


def _norm(acc):
    return pl.pallas_call(
        _norm_body,
        grid=(NP // _ROWB2,),
        in_specs=[pl.BlockSpec((NC, _ROWB2, ACCW), lambda i: (0, i, 0))],
        out_specs=pl.BlockSpec((_ROWB2, HIDDEN), lambda i: (i, 0)),
        out_shape=jax.ShapeDtypeStruct((NP, HIDDEN), jnp.float32),
    )(acc)


def kernel(h, edge_index, WQ_w, WQ_b, WK_w, WK_b, WV_w, WV_b):
    src = edge_index[0].astype(jnp.int32)
    dst = edge_index[1].astype(jnp.int32)
    pad = EP - N_EDGES
    src_p = jnp.concatenate([src, jnp.zeros((pad,), jnp.int32)])
    dst_p = jnp.concatenate([dst, jnp.full((pad,), N_NODES, jnp.int32)])
    ei = jnp.stack([src_p, dst_p])
    bq2 = WQ_b.reshape(NC, HALF)
    bk2 = WK_b.reshape(NC, HALF)
    bv2 = WV_b.reshape(NC, HALF)
    qt, kvt = _qkv(h, WQ_w, bq2, WK_w, bk2, WV_w, bv2)
    zeros = jnp.zeros((NP, ACCW), jnp.float32)
    acc = _edges(qt, kvt, ei, zeros)
    out = _norm(acc)
    return out[:N_NODES].reshape(N_NODES, HEADS, DH)


# speedup vs baseline: 48.8544x; 1.1933x over previous
import dataclasses
import functools

import jax
import jax.numpy as jnp
from jax import lax
from jax.experimental import pallas as pl
from jax.experimental.pallas import tpu as pltpu
from jax.experimental.pallas import tpu_sc as plsc

N_NODES = 10000
N_EDGES = 160000
HIDDEN = 256
HEADS = 8
DH = 32
HALF = 128
ACCW = 144
NC = 2
NS = 16
LANES = 16
CHUNK = 64
EP = 160768
EDGES_PER_TILE = EP // NS
NCHUNK = EDGES_PER_TILE // CHUNK
NP = 10240
ROWS_PER_TILE = NP // NS
INV_SCALE = 1.0 / (DH ** 0.5)

_ROWB = 2000
_ROWB2 = 2048


def _qkv_body(h_ref, wq_ref, bq_ref, wk_ref, bk_ref, wv_ref, bv_ref,
              qt_ref, kvt_ref):
    hb = h_ref[...]

    def proj(w_ref, b_ref, half):
        w = w_ref[half * HALF:(half + 1) * HALF, :]
        r = lax.dot_general(hb, w, (((1,), (1,)), ((), ())),
                            preferred_element_type=jnp.float32)
        return r + b_ref[half]

    for half in range(2):
        qt_ref[half] = proj(wq_ref, bq_ref, half).astype(jnp.bfloat16)
        kvt_ref[half, :, 0:HALF] = proj(wk_ref, bk_ref, half).astype(jnp.bfloat16)
        kvt_ref[half, :, HALF:2 * HALF] = proj(wv_ref, bv_ref, half).astype(jnp.bfloat16)


def _qkv(h, WQ_w, bq2, WK_w, bk2, WV_w, bv2):
    wspec = pl.BlockSpec((HIDDEN, HIDDEN), lambda i: (0, 0))
    bspec = pl.BlockSpec((NC, HALF), lambda i: (0, 0))
    return pl.pallas_call(
        _qkv_body,
        grid=(N_NODES // _ROWB,),
        in_specs=[pl.BlockSpec((_ROWB, HIDDEN), lambda i: (i, 0)),
                  wspec, bspec, wspec, bspec, wspec, bspec],
        out_specs=[pl.BlockSpec((NC, _ROWB, HALF), lambda i: (0, i, 0)),
                   pl.BlockSpec((NC, _ROWB, 2 * HALF), lambda i: (0, i, 0))],
        out_shape=[jax.ShapeDtypeStruct((NC, N_NODES, HALF), jnp.bfloat16),
                   jax.ShapeDtypeStruct((NC, N_NODES, 2 * HALF), jnp.bfloat16)],
    )(h, WQ_w, bq2, WK_w, bk2, WV_w, bv2)


def _edge_body(qt_hbm, kvt_hbm, ei_hbm, zero_hbm, acc_hbm,
               ei_v0, ei_v1, dsc_v0, dsc_v1, gcl_v0, gcl_v1,
               qd_v0, qd_v1, kv_v0, kv_v1, row_v0, row_v1,
               acc_sh, semi0, semi1, semg0, semg1, sems0, sems1):
    c = lax.axis_index("c")
    s = lax.axis_index("s")
    rowbase = s * ROWS_PER_TILE
    lane = lax.iota(jnp.int32, LANES)

    pltpu.sync_copy(zero_hbm.at[pl.ds(rowbase, ROWS_PER_TILE)],
                    acc_sh.at[pl.ds(rowbase, ROWS_PER_TILE)])
    plsc.subcore_barrier()

    EI = (ei_v0, ei_v1)
    DSC = (dsc_v0, dsc_v1)
    GCL = (gcl_v0, gcl_v1)
    QD = (qd_v0, qd_v1)
    KV = (kv_v0, kv_v1)
    ROW = (row_v0, row_v0)
    SEMI = (semi0, semi1)
    SEMG = (semg0, semg1)
    SEMS = (sems0, sems0)

    def idx_copy(k, p):
        base = s * EDGES_PER_TILE + k * CHUNK
        return pltpu.make_async_copy(ei_hbm.at[:, pl.ds(base, CHUNK)],
                                     EI[p], SEMI[p])

    def gather_copies(p):
        return (pltpu.make_async_copy(qt_hbm.at[c].at[GCL[p]], QD[p], SEMG[p]),
                pltpu.make_async_copy(kvt_hbm.at[c].at[EI[p].at[0]], KV[p],
                                      SEMG[p]))

    def scatter_copy(p):
        return pltpu.make_async_copy(ROW[p], acc_sh.at[DSC[p]], SEMS[p])

    def build_gcl(p):
        for i in range(CHUNK // LANES):
            GCL[p][pl.ds(i * LANES, LANES)] = jnp.minimum(
                EI[p][1, pl.ds(i * LANES, LANES)], N_NODES - 1)

    def build_dsc(p):
        for i in range(CHUNK // LANES):
            DSC[p][pl.ds(i * LANES, LANES)] = EI[p][1, pl.ds(i * LANES, LANES)]

    def compute(p):
        qd_v, kv_v, row_v = QD[p], KV[p], ROW[p]

        @plsc.parallel_loop(0, CHUNK, unroll=4)
        def _edge(e):
            den = None
            for hh in range(4):
                qa, qb = plsc.unpack(qd_v[e, pl.ds(hh * 32, 2 * LANES)],
                                     format=plsc.PackFormat.INTERLEAVED)
                ka, kb = plsc.unpack(kv_v[e, pl.ds(hh * 32, 2 * LANES)],
                                     format=plsc.PackFormat.INTERLEAVED)
                score = jnp.sum(qa * ka + qb * kb) * INV_SCALE
                eb = jnp.exp(jnp.broadcast_to(score, (LANES,)))
                va, vb = plsc.unpack(kv_v[e, pl.ds(HALF + hh * 32, 2 * LANES)],
                                     format=plsc.PackFormat.INTERLEAVED)
                row_v[e, pl.ds(hh * 32, LANES)] = eb * va
                row_v[e, pl.ds(hh * 32 + LANES, LANES)] = eb * vb
                masked = jnp.where(lane == hh, eb, 0.0)
                den = masked if den is None else den + masked
            row_v[e, pl.ds(HALF, LANES)] = den

    def process(k, p, q, first, last):
        gather_copies(p)[0].wait()
        gather_copies(p)[1].wait()
        build_dsc(p)
        @pl.when(k + 2 < NCHUNK)
        def _():
            idx_copy(k + 2, p).start()
        @pl.when(k + 1 < NCHUNK)
        def _():
            idx_copy(k + 1, q).wait()
            build_gcl(q)
            for cp_ in gather_copies(q):
                cp_.start()
        if not first:
            scatter_copy(p).wait()
        compute(p)
        scatter_copy(p).start(add=True)

    idx_copy(0, 0).start()
    idx_copy(1, 1).start()
    idx_copy(0, 0).wait()
    build_gcl(0)
    for cp_ in gather_copies(0):
        cp_.start()

    process(0, 0, 1, True, False)
    process(1, 1, 0, False, False)

    @pl.loop(2, NCHUNK - 1, step=2)
    def _chunk(g):
        process(g, 0, 1, False, False)
        process(g + 1, 1, 0, False, False)

    process(NCHUNK - 1, 0, 1, False, True)

    scatter_copy(0).wait()
    plsc.subcore_barrier()
    pltpu.sync_copy(acc_sh.at[pl.ds(rowbase, ROWS_PER_TILE)],
                    acc_hbm.at[c].at[pl.ds(rowbase, ROWS_PER_TILE)])


def _edges(qt, kvt, ei, zeros):
    mesh = plsc.VectorSubcoreMesh(core_axis_name="c", subcore_axis_name="s")
    cp = pltpu.CompilerParams()
    if "needs_layout_passes" in pltpu.CompilerParams.__dataclass_fields__:
        cp = dataclasses.replace(cp, needs_layout_passes=False)
    if "use_tc_tiling_on_sc" in pltpu.CompilerParams.__dataclass_fields__:
        cp = dataclasses.replace(cp, use_tc_tiling_on_sc=False)
    fn = functools.partial(
        pl.kernel,
        mesh=mesh,
        compiler_params=cp,
        out_type=jax.ShapeDtypeStruct((NC, NP, ACCW), jnp.float32),
        scratch_types=[
            pltpu.VMEM((2, CHUNK), jnp.int32),
            pltpu.VMEM((2, CHUNK), jnp.int32),
            pltpu.VMEM((CHUNK,), jnp.int32),
            pltpu.VMEM((CHUNK,), jnp.int32),
            pltpu.VMEM((CHUNK,), jnp.int32),
            pltpu.VMEM((CHUNK,), jnp.int32),
            pltpu.VMEM((CHUNK, HALF), jnp.bfloat16),
            pltpu.VMEM((CHUNK, HALF), jnp.bfloat16),
            pltpu.VMEM((CHUNK, 2 * HALF), jnp.bfloat16),
            pltpu.VMEM((CHUNK, 2 * HALF), jnp.bfloat16),
            pltpu.VMEM((CHUNK, ACCW), jnp.float32),
            pltpu.VMEM((CHUNK, ACCW), jnp.float32),
            pltpu.VMEM_SHARED((NP, ACCW), jnp.float32),
            pltpu.SemaphoreType.DMA,
            pltpu.SemaphoreType.DMA,
            pltpu.SemaphoreType.DMA,
            pltpu.SemaphoreType.DMA,
            pltpu.SemaphoreType.DMA,
            pltpu.SemaphoreType.DMA,
        ],
    )(_edge_body)
    return fn(qt, kvt, ei, zeros)


def _norm_body(acc_ref, out_ref):
    for c in range(NC):
        for hh in range(4):
            numer = acc_ref[c, :, hh * 32:(hh + 1) * 32]
            den = acc_ref[c, :, HALF + hh:HALF + hh + 1]
            out_ref[:, (c * 4 + hh) * 32:(c * 4 + hh + 1) * 32] = (
                numer / jnp.maximum(den, 1e-16))


def _norm(acc):
    return pl.pallas_call(
        _norm_body,
        grid=(NP // _ROWB2,),
        in_specs=[pl.BlockSpec((NC, _ROWB2, ACCW), lambda i: (0, i, 0))],
        out_specs=pl.BlockSpec((_ROWB2, HIDDEN), lambda i: (i, 0)),
        out_shape=jax.ShapeDtypeStruct((NP, HIDDEN), jnp.float32),
    )(acc)


def kernel(h, edge_index, WQ_w, WQ_b, WK_w, WK_b, WV_w, WV_b):
    src = edge_index[0].astype(jnp.int32)
    dst = edge_index[1].astype(jnp.int32)
    pad = EP - N_EDGES
    src_p = jnp.concatenate([src, jnp.zeros((pad,), jnp.int32)])
    dst_p = jnp.concatenate([dst, jnp.full((pad,), N_NODES, jnp.int32)])
    ei = jnp.stack([src_p, dst_p])
    bq2 = WQ_b.reshape(NC, HALF)
    bk2 = WK_b.reshape(NC, HALF)
    bv2 = WV_b.reshape(NC, HALF)
    qt, kvt = _qkv(h, WQ_w, bq2, WK_w, bk2, WV_w, bv2)
    zeros = jnp.zeros((NP, ACCW), jnp.float32)
    acc = _edges(qt, kvt, ei, zeros)
    out = _norm(acc)
    out3 = out[:N_NODES].reshape(N_NODES, HEADS, DH)
    perm = jnp.array([d // 2 if d % 2 == 0 else 16 + d // 2
                      for d in range(DH)], jnp.int32)
    return jnp.take(out3, perm, axis=2)


# speedup vs baseline: 48.8701x; 1.0003x over previous
import dataclasses
import functools

import jax
import jax.numpy as jnp
from jax import lax
from jax.experimental import pallas as pl
from jax.experimental.pallas import tpu as pltpu
from jax.experimental.pallas import tpu_sc as plsc

N_NODES = 10000
N_EDGES = 160000
HIDDEN = 256
HEADS = 8
DH = 32
HALF = 128
ACCW = 144
NC = 2
NS = 16
LANES = 16
CHUNK = 64
EP = 160768
EDGES_PER_TILE = EP // NS
NCHUNK = EDGES_PER_TILE // CHUNK
NP = 10240
ROWS_PER_TILE = NP // NS
INV_SCALE = 1.0 / (DH ** 0.5)

_ROWB = 2000
_ROWB2 = 2048


def _qkv_body(h_ref, wq_ref, bq_ref, wk_ref, bk_ref, wv_ref, bv_ref,
              qt_ref, kvt_ref):
    hb = h_ref[...]

    def proj(w_ref, b_ref, half):
        w = w_ref[half * HALF:(half + 1) * HALF, :]
        r = lax.dot_general(hb, w, (((1,), (1,)), ((), ())),
                            preferred_element_type=jnp.float32)
        return r + b_ref[half]

    for half in range(2):
        qt_ref[half] = proj(wq_ref, bq_ref, half).astype(jnp.bfloat16)
        kvt_ref[half, :, 0:HALF] = proj(wk_ref, bk_ref, half).astype(jnp.bfloat16)
        kvt_ref[half, :, HALF:2 * HALF] = proj(wv_ref, bv_ref, half).astype(jnp.bfloat16)


def _qkv(h, WQ_w, bq2, WK_w, bk2, WV_w, bv2):
    wspec = pl.BlockSpec((HIDDEN, HIDDEN), lambda i: (0, 0))
    bspec = pl.BlockSpec((NC, HALF), lambda i: (0, 0))
    return pl.pallas_call(
        _qkv_body,
        grid=(N_NODES // _ROWB,),
        in_specs=[pl.BlockSpec((_ROWB, HIDDEN), lambda i: (i, 0)),
                  wspec, bspec, wspec, bspec, wspec, bspec],
        out_specs=[pl.BlockSpec((NC, _ROWB, HALF), lambda i: (0, i, 0)),
                   pl.BlockSpec((NC, _ROWB, 2 * HALF), lambda i: (0, i, 0))],
        out_shape=[jax.ShapeDtypeStruct((NC, N_NODES, HALF), jnp.bfloat16),
                   jax.ShapeDtypeStruct((NC, N_NODES, 2 * HALF), jnp.bfloat16)],
    )(h, WQ_w, bq2, WK_w, bk2, WV_w, bv2)


def _edge_body(qt_hbm, kvt_hbm, ei_hbm, zero_hbm, acc_hbm,
               ei_v0, ei_v1, dsc_v0, dsc_v1, gcl_v0, gcl_v1,
               qd_v0, qd_v1, kv_v0, kv_v1, row_v0,
               acc_sh, semi0, semi1, semg0, semg1, sems0, sems1):
    c = lax.axis_index("c")
    s = lax.axis_index("s")
    rowbase = s * ROWS_PER_TILE
    lane = lax.iota(jnp.int32, LANES)

    pltpu.sync_copy(zero_hbm.at[pl.ds(rowbase, ROWS_PER_TILE)],
                    acc_sh.at[pl.ds(rowbase, ROWS_PER_TILE)])
    plsc.subcore_barrier()

    EI = (ei_v0, ei_v1)
    DSC = (dsc_v0, dsc_v1)
    GCL = (gcl_v0, gcl_v1)
    QD = (qd_v0, qd_v1)
    KV = (kv_v0, kv_v1)
    ROW = (row_v0, row_v0)
    SEMI = (semi0, semi1)
    SEMG = (semg0, semg1)
    SEMS = (sems0, sems0)

    def idx_copy(k, p):
        base = s * EDGES_PER_TILE + k * CHUNK
        return pltpu.make_async_copy(ei_hbm.at[:, pl.ds(base, CHUNK)],
                                     EI[p], SEMI[p])

    def gather_copies(p):
        return (pltpu.make_async_copy(qt_hbm.at[c].at[GCL[p]], QD[p], SEMG[p]),
                pltpu.make_async_copy(kvt_hbm.at[c].at[EI[p].at[0]], KV[p],
                                      SEMG[p]))

    def scatter_copy(p):
        return pltpu.make_async_copy(ROW[p], acc_sh.at[DSC[p]], SEMS[p])

    def build_gcl(p):
        for i in range(CHUNK // LANES):
            GCL[p][pl.ds(i * LANES, LANES)] = jnp.minimum(
                EI[p][1, pl.ds(i * LANES, LANES)], N_NODES - 1)

    def build_dsc(p):
        for i in range(CHUNK // LANES):
            DSC[p][pl.ds(i * LANES, LANES)] = EI[p][1, pl.ds(i * LANES, LANES)]

    def compute(p):
        qd_v, kv_v, row_v = QD[p], KV[p], ROW[p]

        @plsc.parallel_loop(0, CHUNK, unroll=4)
        def _edge(e):
            den = None
            for hh in range(4):
                qa, qb = plsc.unpack(qd_v[e, pl.ds(hh * 32, 2 * LANES)],
                                     format=plsc.PackFormat.INTERLEAVED)
                ka, kb = plsc.unpack(kv_v[e, pl.ds(hh * 32, 2 * LANES)],
                                     format=plsc.PackFormat.INTERLEAVED)
                score = jnp.sum(qa * ka + qb * kb) * INV_SCALE
                eb = jnp.exp(jnp.broadcast_to(score, (LANES,)))
                va, vb = plsc.unpack(kv_v[e, pl.ds(HALF + hh * 32, 2 * LANES)],
                                     format=plsc.PackFormat.INTERLEAVED)
                row_v[e, pl.ds(hh * 32, LANES)] = eb * va
                row_v[e, pl.ds(hh * 32 + LANES, LANES)] = eb * vb
                masked = jnp.where(lane == hh, eb, 0.0)
                den = masked if den is None else den + masked
            row_v[e, pl.ds(HALF, LANES)] = den

    def process(k, p, q, first, last):
        gather_copies(p)[0].wait()
        gather_copies(p)[1].wait()
        build_dsc(p)
        @pl.when(k + 2 < NCHUNK)
        def _():
            idx_copy(k + 2, p).start()
        @pl.when(k + 1 < NCHUNK)
        def _():
            idx_copy(k + 1, q).wait()
            build_gcl(q)
            for cp_ in gather_copies(q):
                cp_.start()
        if not first:
            scatter_copy(p).wait()
        compute(p)
        scatter_copy(p).start(add=True)

    idx_copy(0, 0).start()
    idx_copy(1, 1).start()
    idx_copy(0, 0).wait()
    build_gcl(0)
    for cp_ in gather_copies(0):
        cp_.start()

    process(0, 0, 1, True, False)
    process(1, 1, 0, False, False)

    @pl.loop(2, NCHUNK - 1, step=2)
    def _chunk(g):
        process(g, 0, 1, False, False)
        process(g + 1, 1, 0, False, False)

    process(NCHUNK - 1, 0, 1, False, True)

    scatter_copy(0).wait()
    plsc.subcore_barrier()
    pltpu.sync_copy(acc_sh.at[pl.ds(rowbase, ROWS_PER_TILE)],
                    acc_hbm.at[c].at[pl.ds(rowbase, ROWS_PER_TILE)])


def _edges(qt, kvt, ei, zeros):
    mesh = plsc.VectorSubcoreMesh(core_axis_name="c", subcore_axis_name="s")
    cp = pltpu.CompilerParams()
    if "needs_layout_passes" in pltpu.CompilerParams.__dataclass_fields__:
        cp = dataclasses.replace(cp, needs_layout_passes=False)
    if "use_tc_tiling_on_sc" in pltpu.CompilerParams.__dataclass_fields__:
        cp = dataclasses.replace(cp, use_tc_tiling_on_sc=False)
    fn = functools.partial(
        pl.kernel,
        mesh=mesh,
        compiler_params=cp,
        out_type=jax.ShapeDtypeStruct((NC, NP, ACCW), jnp.float32),
        scratch_types=[
            pltpu.VMEM((2, CHUNK), jnp.int32),
            pltpu.VMEM((2, CHUNK), jnp.int32),
            pltpu.VMEM((CHUNK,), jnp.int32),
            pltpu.VMEM((CHUNK,), jnp.int32),
            pltpu.VMEM((CHUNK,), jnp.int32),
            pltpu.VMEM((CHUNK,), jnp.int32),
            pltpu.VMEM((CHUNK, HALF), jnp.bfloat16),
            pltpu.VMEM((CHUNK, HALF), jnp.bfloat16),
            pltpu.VMEM((CHUNK, 2 * HALF), jnp.bfloat16),
            pltpu.VMEM((CHUNK, 2 * HALF), jnp.bfloat16),
            pltpu.VMEM((CHUNK, ACCW), jnp.float32),
            pltpu.VMEM_SHARED((NP, ACCW), jnp.float32),
            pltpu.SemaphoreType.DMA,
            pltpu.SemaphoreType.DMA,
            pltpu.SemaphoreType.DMA,
            pltpu.SemaphoreType.DMA,
            pltpu.SemaphoreType.DMA,
            pltpu.SemaphoreType.DMA,
        ],
    )(_edge_body)
    return fn(qt, kvt, ei, zeros)


def _norm_body(acc_ref, out_ref):
    for c in range(NC):
        for hh in range(4):
            numer = acc_ref[c, :, hh * 32:(hh + 1) * 32]
            den = acc_ref[c, :, HALF + hh:HALF + hh + 1]
            out_ref[:, (c * 4 + hh) * 32:(c * 4 + hh + 1) * 32] = (
                numer / jnp.maximum(den, 1e-16))


def _norm(acc):
    return pl.pallas_call(
        _norm_body,
        grid=(NP // _ROWB2,),
        in_specs=[pl.BlockSpec((NC, _ROWB2, ACCW), lambda i: (0, i, 0))],
        out_specs=pl.BlockSpec((_ROWB2, HIDDEN), lambda i: (i, 0)),
        out_shape=jax.ShapeDtypeStruct((NP, HIDDEN), jnp.float32),
    )(acc)


def kernel(h, edge_index, WQ_w, WQ_b, WK_w, WK_b, WV_w, WV_b):
    src = edge_index[0].astype(jnp.int32)
    dst = edge_index[1].astype(jnp.int32)
    pad = EP - N_EDGES
    src_p = jnp.concatenate([src, jnp.zeros((pad,), jnp.int32)])
    dst_p = jnp.concatenate([dst, jnp.full((pad,), N_NODES, jnp.int32)])
    ei = jnp.stack([src_p, dst_p])
    bq2 = WQ_b.reshape(NC, HALF)
    bk2 = WK_b.reshape(NC, HALF)
    bv2 = WV_b.reshape(NC, HALF)
    qt, kvt = _qkv(h, WQ_w, bq2, WK_w, bk2, WV_w, bv2)
    zeros = jnp.zeros((NP, ACCW), jnp.float32)
    acc = _edges(qt, kvt, ei, zeros)
    out = _norm(acc)
    out3 = out[:N_NODES].reshape(N_NODES, HEADS, DH)
    perm = jnp.array([d // 2 if d % 2 == 0 else 16 + d // 2
                      for d in range(DH)], jnp.int32)
    return jnp.take(out3, perm, axis=2)
